# baseline (speedup 1.0000x reference)
import functools

import jax
import jax.numpy as jnp
from jax import lax
from jax.experimental import pallas as pl
from jax.experimental.pallas import tpu as pltpu
from jax.experimental.pallas import tpu_sc as plsc

_SEQ = 16384
_T = 50
_B = _SEQ * _T
_D = 64
_NC = 2
_NS = 16
_NW = _NC * _NS
_BPW = _B // _NW
_SPW = _BPW // _T
_SCH = 4
_CH = _SCH * _T
_NCH = _BPW // _CH
_CHP = 208
_STG = 3200
_CPS = _STG // _CH
_NSTG = _BPW // _STG


def _embedding_gather(idx, table2):
    mesh = plsc.VectorSubcoreMesh(core_axis_name="c", subcore_axis_name="s")
    nrows = table2.shape[0]

    @functools.partial(
        pl.kernel,
        mesh=mesh,
        out_type=jax.ShapeDtypeStruct((_SEQ, _T, _D), jnp.float32),
        scratch_types=[
            pltpu.VMEM((_STG + 24,), jnp.int32),
            pltpu.VMEM((_STG + 24,), jnp.int32),
            pltpu.VMEM((_CHP,), jnp.int32),
            pltpu.VMEM((_CHP,), jnp.int32),
            pltpu.VMEM((_CHP, 2 * _D), jnp.float32),
            pltpu.VMEM((_CHP, 2 * _D), jnp.float32),
            pltpu.VMEM((_SCH, _T, _D), jnp.float32),
            pltpu.VMEM((_SCH, _T, _D), jnp.float32),
            pltpu.SemaphoreType.DMA((2,)),
            pltpu.SemaphoreType.DMA((2,)),
            pltpu.SemaphoreType.DMA((2,)),
        ],
    )
    def k(idx_hbm, table_hbm, out_hbm, istg0, istg1, pls0, pls1,
          gb0, gb1, ob0, ob1, isem, gsem, wsem):
        istg = (istg0, istg1)
        plist = (pls0, pls1)
        gbuf = (gb0, gb1)
        obuf = (ob0, ob1)
        wid = lax.axis_index("s") * _NC + lax.axis_index("c")
        base = wid * _BPW
        seq_base = wid * _SPW

        def idx_desc(s, e):
            return pltpu.make_async_copy(
                idx_hbm.at[pl.ds(base + s * _STG, _STG)],
                istg[e].at[pl.ds(0, _STG)],
                isem.at[e],
            )

        def gather_desc(b):
            return pltpu.make_async_copy(
                table_hbm.at[plist[b]],
                gbuf[b],
                gsem.at[b],
            )

        def wb_desc(c, b):
            return pltpu.make_async_copy(
                obuf[b],
                out_hbm.at[pl.ds(seq_base + c * _SCH, _SCH)],
                wsem.at[b],
            )

        def compute_plist(e, j, b):
            for g in range(_CHP // 16):
                v = istg[e][pl.ds(j * _CH + g * 16, 16)]
                p = lax.shift_right_logical(v, 1)
                plist[b][pl.ds(g * 16, 16)] = lax.min(
                    p, jnp.int32(nrows - 1)
                )

        def extract(e, j, b):
            @pl.loop(0, _CH, unroll=2)
            def _row(r):
                tokv = istg[e][pl.ds(j * _CH + r, 16)]
                tok = tokv[0]
                cb = lax.bitwise_and(tok, 1) * _D
                s_ = r // _T
                t_ = r % _T
                for kq in range(_D // 16):
                    obuf[b][s_, t_, pl.ds(kq * 16, 16)] = (
                        gbuf[b][r, pl.ds(cb + kq * 16, 16)]
                    )

        idx_desc(0, 0).start()

        @pl.loop(0, _NSTG // 2)
        def _spair(sp):
            for e in range(2):
                s = sp * 2 + e
                idx_desc(s, e).wait()

                @pl.when(s < _NSTG - 1)
                def _ipre():
                    idx_desc(s + 1, 1 - e).start()

                @pl.loop(0, _CPS // 2)
                def _jpair(jj):
                    for b in range(2):
                        j = jj * 2 + b
                        c = s * _CPS + j
                        compute_plist(e, j, b)

                        @pl.when(jnp.logical_or(s > 0, jj > 0))
                        def _wdrain():
                            wb_desc(c - 2, b).wait()

                        gather_desc(b).start()
                    for b in range(2):
                        j = jj * 2 + b
                        c = s * _CPS + j
                        gather_desc(b).wait()
                        extract(e, j, b)
                        wb_desc(c, b).start()

        for b in range(2):
            wb_desc(_NCH - 2 + b, b).wait()

    return k(idx, table2)


def kernel(tokens, table):
    idx = tokens.reshape(-1).astype(jnp.int32)
    table2 = table.reshape(table.shape[0] // 2, 2 * _D)
    return _embedding_gather(idx, table2)


# speedup vs baseline: 1.4090x; 1.4090x over previous
import functools

import jax
import jax.numpy as jnp
from jax import lax
from jax.experimental import pallas as pl
from jax.experimental.pallas import tpu as pltpu
from jax.experimental.pallas import tpu_sc as plsc

_SEQ = 16384
_T = 50
_B = _SEQ * _T
_D = 64
_NC = 2
_NS = 16
_NW = _NC * _NS
_BPW = _B // _NW
_PPW = _BPW // 2
_CH = 128
_NCH = _PPW // _CH
_NSLOT = 4
_NROUNDS = _NCH // _NSLOT


def _embedding_gather(idx, table):
    mesh = plsc.VectorSubcoreMesh(core_axis_name="c", subcore_axis_name="s")

    @functools.partial(
        pl.kernel,
        mesh=mesh,
        compiler_params=pltpu.CompilerParams(
            use_tc_tiling_on_sc=False, needs_layout_passes=False
        ),
        out_type=jax.ShapeDtypeStruct((_B // 2, 2 * _D), jnp.float32),
        scratch_types=[
            pltpu.VMEM((_BPW,), jnp.int32),
            pltpu.VMEM((_PPW,), jnp.int32),
            pltpu.VMEM((_PPW,), jnp.int32),
            pltpu.VMEM((_NSLOT, 2, _CH, _D), jnp.float32),
            pltpu.SemaphoreType.DMA((_NSLOT,)),
            pltpu.SemaphoreType.DMA((_NSLOT,)),
        ],
    )
    def k(idx_hbm, table_hbm, out_hbm, idx_v, ev_v, od_v, bufs, gsem, wsem):
        wid = lax.axis_index("s") * _NC + lax.axis_index("c")
        base = wid * _BPW
        prow0 = wid * _PPW
        pltpu.sync_copy(idx_hbm.at[pl.ds(base, _BPW)], idx_v)

        lanes = lax.iota(jnp.int32, 16)

        @pl.loop(0, _PPW // 16, unroll=4)
        def _deint(g):
            pos = (g * 16 + lanes) * 2
            ev_v[pl.ds(g * 16, 16)] = plsc.load_gather(idx_v, [pos])
            od_v[pl.ds(g * 16, 16)] = plsc.load_gather(idx_v, [pos + 1])

        halves = (ev_v, od_v)

        def gather_desc(c, b, h):
            return pltpu.make_async_copy(
                table_hbm.at[halves[h].at[pl.ds(c * _CH, _CH)]],
                bufs.at[b].at[h],
                gsem.at[b],
            )

        def wb_desc(c, b, h):
            return pltpu.make_async_copy(
                bufs.at[b].at[h],
                out_hbm.at[pl.ds(prow0 + c * _CH, _CH), pl.ds(h * _D, _D)],
                wsem.at[b],
            )

        for b in range(_NSLOT):
            for h in range(2):
                gather_desc(b, b, h).start()

        @pl.loop(0, _NROUNDS)
        def _round(g):
            c0 = g * _NSLOT
            for b in range(_NSLOT):
                for h in range(2):
                    gather_desc(c0 + b, b, h).wait()
                    wb_desc(c0 + b, b, h).start()

            @pl.when(g < _NROUNDS - 1)
            def _prefetch():
                for b in range(_NSLOT):
                    for h in range(2):
                        wb_desc(c0 + b, b, h).wait()
                        gather_desc(c0 + _NSLOT + b, b, h).start()

        cl = (_NROUNDS - 1) * _NSLOT
        for b in range(_NSLOT):
            for h in range(2):
                wb_desc(cl + b, b, h).wait()

    return k(idx, table)


def kernel(tokens, table):
    idx = tokens.reshape(-1).astype(jnp.int32)
    out2 = _embedding_gather(idx, table)
    return out2.reshape(_SEQ, _T, _D)


# speedup vs baseline: 1.4155x; 1.0046x over previous
import functools

import jax
import jax.numpy as jnp
from jax import lax
from jax.experimental import pallas as pl
from jax.experimental.pallas import tpu as pltpu
from jax.experimental.pallas import tpu_sc as plsc

_SEQ = 16384
_T = 50
_B = _SEQ * _T
_D = 64
_NC = 2
_NS = 16
_NW = _NC * _NS
_BPW = _B // _NW
_CH = 200
_SCH = _CH // _T
_NCH = _BPW // _CH
_NSLOT = 8
_NROUNDS = _NCH // _NSLOT


def _embedding_gather(idx, table):
    mesh = plsc.VectorSubcoreMesh(core_axis_name="c", subcore_axis_name="s")

    @functools.partial(
        pl.kernel,
        mesh=mesh,
        compiler_params=pltpu.CompilerParams(use_tc_tiling_on_sc=False),
        out_type=jax.ShapeDtypeStruct((_SEQ, _T, _D), jnp.float32),
        scratch_types=[
            pltpu.VMEM((_BPW,), jnp.int32),
            pltpu.VMEM((_NSLOT, _CH, _D), jnp.float32),
            pltpu.SemaphoreType.DMA((_NSLOT,)),
            pltpu.SemaphoreType.DMA((_NSLOT,)),
        ],
    )
    def k(idx_hbm, table_hbm, out_hbm, idx_v, bufs, gsem, wsem):
        wid = lax.axis_index("s") * _NC + lax.axis_index("c")
        base = wid * _BPW
        seq_base = wid * (_BPW // _T)
        pltpu.sync_copy(idx_hbm.at[pl.ds(base, _BPW)], idx_v)

        def gather_desc(c, b):
            return pltpu.make_async_copy(
                table_hbm.at[idx_v.at[pl.ds(c * _CH, _CH)]],
                bufs.at[b],
                gsem.at[b],
            )

        def wb_desc(c, b, s):
            return pltpu.make_async_copy(
                bufs.at[b].at[pl.ds(s * _T, _T)],
                out_hbm.at[seq_base + c * _SCH + s],
                wsem.at[b],
            )

        def wb_start(c, b):
            for s in range(_SCH):
                wb_desc(c, b, s).start()

        def wb_wait(c, b):
            for s in range(_SCH):
                wb_desc(c, b, s).wait()

        for b in range(_NSLOT):
            gather_desc(b, b).start()

        @pl.loop(0, _NROUNDS)
        def _round(g):
            c0 = g * _NSLOT
            for b in range(_NSLOT):
                gather_desc(c0 + b, b).wait()
                wb_start(c0 + b, b)

            @pl.when(g < _NROUNDS - 1)
            def _prefetch():
                for b in range(_NSLOT):
                    wb_wait(c0 + b, b)
                    gather_desc(c0 + _NSLOT + b, b).start()

        cl = (_NROUNDS - 1) * _NSLOT
        for b in range(_NSLOT):
            wb_wait(cl + b, b)

    return k(idx, table)


def kernel(tokens, table):
    idx = tokens.reshape(-1).astype(jnp.int32)
    return _embedding_gather(idx, table)
